# baseline (speedup 1.0000x reference)
import functools

import jax
import jax.numpy as jnp
from jax.experimental import pallas as pl
from jax.experimental.pallas import tpu as pltpu

_CH = 512
_LA = 8
_NBUF = _LA + 1


def _softplus(x):
    return jnp.maximum(x, 0.0) + jnp.log1p(jnp.exp(-jnp.abs(x)))


def _read_head_kernel(emb_ref, ws_ref, bs_ref, wsh_ref, bsh_ref, mem_hbm,
                      out_ref, bufs, sems, acc, zsum, mrun):
    i = pl.program_id(0)
    nch = pl.num_programs(0)

    @pl.when(i == 0)
    def _prologue():
        acc[...] = jnp.zeros_like(acc)
        zsum[...] = jnp.zeros_like(zsum)
        mrun[...] = jnp.full_like(mrun, -1e30)
        for c in range(_LA):
            pltpu.make_async_copy(
                mem_hbm.at[pl.ds(c * _CH, _CH), :], bufs.at[c], sems.at[c]
            ).start()

    @pl.when(i + _LA < nch)
    def _prefetch():
        c = i + _LA
        slot = jax.lax.rem(c, _NBUF)
        pltpu.make_async_copy(
            mem_hbm.at[pl.ds(c * _CH, _CH), :], bufs.at[slot], sems.at[slot]
        ).start()

    slot = jax.lax.rem(i, _NBUF)
    pltpu.make_async_copy(
        mem_hbm.at[pl.ds(i * _CH, _CH), :], bufs.at[slot], sems.at[slot]
    ).wait()

    coding = emb_ref[...]
    s_lin = jnp.sum(coding * ws_ref[...], axis=1, keepdims=True) + bs_ref[0, 0]
    sh_lin = jnp.sum(coding * wsh_ref[...], axis=1, keepdims=True) + bsh_ref[0, 0]
    temp = _softplus(s_lin) * (1.0 + _softplus(sh_lin))
    knorm = jnp.sqrt(jnp.sum(coding * coding, axis=1, keepdims=True))
    key_n = coding / (knorm + 1e-8)

    mem = bufs[slot]
    raw = jax.lax.dot_general(key_n, mem, (((1,), (1,)), ((), ())),
                              preferred_element_type=jnp.float32)
    ones = jnp.ones((1, key_n.shape[1]), dtype=jnp.float32)
    nsq = jax.lax.dot_general(ones, mem * mem, (((1,), (1,)), ((), ())),
                              preferred_element_type=jnp.float32)
    inv = 1.0 / (jnp.sqrt(nsq) + 1e-8)
    logits = temp * (raw * inv)

    m_old = mrun[...]
    m_new = jnp.maximum(m_old, jnp.max(logits, axis=1, keepdims=True))
    alpha = jnp.exp(m_old - m_new)
    p = jnp.exp(logits - m_new)
    zsum[...] = zsum[...] * alpha + jnp.sum(p, axis=1, keepdims=True)
    acc[...] = acc[...] * alpha + jnp.dot(p, mem, preferred_element_type=jnp.float32)
    mrun[...] = m_new

    @pl.when(i == nch - 1)
    def _fin():
        out_ref[...] = acc[...] / zsum[...]


@functools.partial(jax.jit, static_argnames=())
def kernel(embeddings, memory, W_strength, b_strength, W_sharpen, b_sharpen):
    B = embeddings.shape[0]
    N = memory.shape[0]
    D = memory.shape[1] * memory.shape[2] * memory.shape[3]
    emb = embeddings.reshape(B, D)
    mem = memory.reshape(N, D)
    ws = W_strength.reshape(1, D)
    wsh = W_sharpen.reshape(1, D)
    bs = b_strength.reshape(1, 1)
    bsh = b_sharpen.reshape(1, 1)
    nch = N // _CH

    return pl.pallas_call(
        _read_head_kernel,
        grid=(nch,),
        in_specs=[
            pl.BlockSpec((B, D), lambda i: (0, 0)),
            pl.BlockSpec((1, D), lambda i: (0, 0)),
            pl.BlockSpec((1, 1), lambda i: (0, 0)),
            pl.BlockSpec((1, D), lambda i: (0, 0)),
            pl.BlockSpec((1, 1), lambda i: (0, 0)),
            pl.BlockSpec(memory_space=pl.ANY),
        ],
        out_specs=pl.BlockSpec((B, D), lambda i: (0, 0)),
        out_shape=jax.ShapeDtypeStruct((B, D), jnp.float32),
        scratch_shapes=[
            pltpu.VMEM((_NBUF, _CH, D), jnp.float32),
            pltpu.SemaphoreType.DMA((_NBUF,)),
            pltpu.VMEM((B, D), jnp.float32),
            pltpu.VMEM((B, 1), jnp.float32),
            pltpu.VMEM((B, 1), jnp.float32),
        ],
        compiler_params=pltpu.CompilerParams(
            dimension_semantics=("arbitrary",),
        ),
    )(emb, ws, bs, wsh, bsh, mem)


# speedup vs baseline: 3.4079x; 3.4079x over previous
import functools

import jax
import jax.numpy as jnp
from jax.experimental import pallas as pl
from jax.experimental.pallas import tpu as pltpu

_BLK = 2048


def _softplus(x):
    return jnp.maximum(x, 0.0) + jnp.log1p(jnp.exp(-jnp.abs(x)))


def _read_head_kernel(embt_ref, ws_ref, bs_ref, wsh_ref, bsh_ref, memt_ref,
                      out_ref, acc, zsum, mrun):
    i = pl.program_id(0)
    nb = pl.num_programs(0)

    @pl.when(i == 0)
    def _init():
        acc[...] = jnp.zeros_like(acc)
        zsum[...] = jnp.zeros_like(zsum)
        mrun[...] = jnp.full_like(mrun, -1e30)

    embt = embt_ref[...]
    s_row = jax.lax.dot_general(ws_ref[...], embt, (((1,), (0,)), ((), ())),
                                preferred_element_type=jnp.float32) + bs_ref[0, 0]
    sh_row = jax.lax.dot_general(wsh_ref[...], embt, (((1,), (0,)), ((), ())),
                                 preferred_element_type=jnp.float32) + bsh_ref[0, 0]
    temp_row = _softplus(s_row) * (1.0 + _softplus(sh_row))
    ksq_row = jnp.sum(embt * embt, axis=0, keepdims=True)
    tscale_row = temp_row / (jnp.sqrt(ksq_row) + 1e-8)
    nb_b = embt.shape[1]
    eye = (jax.lax.broadcasted_iota(jnp.int32, (nb_b, nb_b), 0)
           == jax.lax.broadcasted_iota(jnp.int32, (nb_b, nb_b), 1))
    tscale = jnp.sum(jnp.where(eye, tscale_row, 0.0), axis=1, keepdims=True)

    memt = memt_ref[...]
    raw = jax.lax.dot_general(embt, memt, (((0,), (0,)), ((), ())),
                              preferred_element_type=jnp.float32)
    nsq = jnp.sum(memt * memt, axis=0, keepdims=True)
    inv = 1.0 / (jnp.sqrt(nsq) + 1e-8)
    logits = tscale * (raw * inv)

    m_old = mrun[...]
    m_new = jnp.maximum(m_old, jnp.max(logits, axis=1, keepdims=True))
    alpha = jnp.exp(m_old - m_new)
    p = jnp.exp(logits - m_new)
    zsum[...] = zsum[...] * alpha + jnp.sum(p, axis=1, keepdims=True)
    acc[...] = acc[...] * alpha + jax.lax.dot_general(
        p, memt, (((1,), (1,)), ((), ())), preferred_element_type=jnp.float32)
    mrun[...] = m_new

    @pl.when(i == nb - 1)
    def _fin():
        out_ref[...] = acc[...] / zsum[...]


@functools.partial(jax.jit, static_argnames=())
def kernel(embeddings, memory, W_strength, b_strength, W_sharpen, b_sharpen):
    B = embeddings.shape[0]
    N = memory.shape[0]
    D = memory.shape[1] * memory.shape[2] * memory.shape[3]
    embt = embeddings.transpose(1, 2, 3, 0).reshape(D, B)
    memt = memory.transpose(1, 2, 3, 0).reshape(D, N)
    ws = W_strength.reshape(1, D)
    wsh = W_sharpen.reshape(1, D)
    bs = b_strength.reshape(1, 1)
    bsh = b_sharpen.reshape(1, 1)
    nb = N // _BLK

    return pl.pallas_call(
        _read_head_kernel,
        grid=(nb,),
        in_specs=[
            pl.BlockSpec((D, B), lambda i: (0, 0)),
            pl.BlockSpec((1, D), lambda i: (0, 0)),
            pl.BlockSpec((1, 1), lambda i: (0, 0)),
            pl.BlockSpec((1, D), lambda i: (0, 0)),
            pl.BlockSpec((1, 1), lambda i: (0, 0)),
            pl.BlockSpec((D, _BLK), lambda i: (0, i)),
        ],
        out_specs=pl.BlockSpec((B, D), lambda i: (0, 0)),
        out_shape=jax.ShapeDtypeStruct((B, D), jnp.float32),
        scratch_shapes=[
            pltpu.VMEM((B, D), jnp.float32),
            pltpu.VMEM((B, 1), jnp.float32),
            pltpu.VMEM((B, 1), jnp.float32),
        ],
        compiler_params=pltpu.CompilerParams(
            dimension_semantics=("arbitrary",),
        ),
    )(embt, ws, bs, wsh, bsh, memt)
